# baseline (speedup 1.0000x reference)
import jax
import jax.numpy as jnp
from jax.experimental import pallas as pl
from jax.experimental.pallas import tpu as pltpu

HIDDEN = 128
N_EDGES = 320000
N_X = 10000
GRID = 25
HALF_BLOCKS = GRID
BE = N_EDGES // (2 * GRID)
BX = N_X // GRID


def _tree_sum8(a):
    rows = a.shape[0]
    if rows > 128 and rows % 128 == 0:
        a = a.reshape(rows // 128, 128, HIDDEN).sum(axis=0)
        rows = 128
    while rows > 8 and rows % 16 == 0:
        rows //= 2
        a = a[:rows] + a[rows:]
    if rows > 8:
        a = a.reshape(rows // 8, 8, HIDDEN).sum(axis=0)
    return a


def _gb_kernel(x_ref, ea_ref, eb_ref, g_ref, w1_ref, b1_ref, w2_ref, b2_ref,
               gamma_ref, beta_ref, out_ref, acc_ref):
    i = pl.program_id(0)

    @pl.when(i == 0)
    def _init():
        acc_ref[...] = jnp.zeros_like(acc_ref)

    @pl.when(i == 0)
    def _sum_x():
        xs = x_ref[...].reshape(N_X // 200, 200, HIDDEN).sum(axis=0)
        acc_ref[0:8, :] += _tree_sum8(xs)

    acc_ref[8:16, :] += _tree_sum8(ea_ref[...])
    acc_ref[16:24, :] += _tree_sum8(eb_ref[...])

    @pl.when(i == GRID - 1)
    def _finish():
        sn = jnp.sum(acc_ref[0:8, :], axis=0, keepdims=True)
        se = jnp.sum(acc_ref[8:16, :] + acc_ref[16:24, :], axis=0, keepdims=True)
        g = g_ref[...]
        h = (jnp.dot(sn, w1_ref[0:HIDDEN, :], preferred_element_type=jnp.float32)
             + jnp.dot(se, w1_ref[HIDDEN:2 * HIDDEN, :], preferred_element_type=jnp.float32)
             + jnp.dot(g, w1_ref[2 * HIDDEN:3 * HIDDEN, :], preferred_element_type=jnp.float32)
             + b1_ref[...])
        h = jnp.maximum(h, 0.0)
        out = jnp.dot(h, w2_ref[...], preferred_element_type=jnp.float32) + b2_ref[...]
        mean = jnp.mean(out, axis=-1, keepdims=True)
        var = jnp.mean((out - mean) ** 2, axis=-1, keepdims=True)
        out_ref[...] = ((out - mean) * jax.lax.rsqrt(var + 1e-5)
                        * gamma_ref[...] + beta_ref[...])


def kernel(x, edge_attr_updated, global_attr, W1, b1, W2, b2, gamma, beta):
    b1r = b1.reshape(1, HIDDEN)
    b2r = b2.reshape(1, HIDDEN)
    gammar = gamma.reshape(1, HIDDEN)
    betar = beta.reshape(1, HIDDEN)

    const = lambda i: (0, 0)
    return pl.pallas_call(
        _gb_kernel,
        grid=(GRID,),
        in_specs=[
            pl.BlockSpec((N_X, HIDDEN), const),
            pl.BlockSpec((BE, HIDDEN), lambda i: (i, 0)),
            pl.BlockSpec((BE, HIDDEN), lambda i: (i + HALF_BLOCKS, 0)),
            pl.BlockSpec((1, HIDDEN), const),
            pl.BlockSpec((3 * HIDDEN, HIDDEN), const),
            pl.BlockSpec((1, HIDDEN), const),
            pl.BlockSpec((HIDDEN, HIDDEN), const),
            pl.BlockSpec((1, HIDDEN), const),
            pl.BlockSpec((1, HIDDEN), const),
            pl.BlockSpec((1, HIDDEN), const),
        ],
        out_specs=pl.BlockSpec((1, HIDDEN), const),
        out_shape=jax.ShapeDtypeStruct((1, HIDDEN), jnp.float32),
        scratch_shapes=[pltpu.VMEM((24, HIDDEN), jnp.float32)],
        compiler_params=pltpu.CompilerParams(
            dimension_semantics=("arbitrary",),
        ),
    )(x, edge_attr_updated, edge_attr_updated, global_attr, W1, b1r, W2,
      b2r, gammar, betar)


# speedup vs baseline: 1.0065x; 1.0065x over previous
import jax
import jax.numpy as jnp
from jax.experimental import pallas as pl
from jax.experimental.pallas import tpu as pltpu

HIDDEN = 128
N_EDGES = 320000
N_X = 10000
GRID = 25
HALF_BLOCKS = GRID
BE = N_EDGES // (2 * GRID)
BX = N_X // GRID


def _tree_sum8(a):
    rows = a.shape[0]
    if rows > 128 and rows % 128 == 0:
        a = a.reshape(rows // 128, 128, HIDDEN).sum(axis=0)
        rows = 128
    while rows > 8 and rows % 16 == 0:
        rows //= 2
        a = a[:rows] + a[rows:]
    if rows > 8:
        a = a.reshape(rows // 8, 8, HIDDEN).sum(axis=0)
    return a


def _gb_kernel(x_ref, ea_ref, eb_ref, g_ref, w1_ref, b1_ref, w2_ref, b2_ref,
               gamma_ref, beta_ref, out_ref, acc_ref):
    i = pl.program_id(0)

    @pl.when(i == 0)
    def _init():
        acc_ref[...] = jnp.zeros_like(acc_ref)

    acc_ref[0:8, :] += _tree_sum8(x_ref[...])
    acc_ref[8:16, :] += _tree_sum8(ea_ref[...])
    acc_ref[16:24, :] += _tree_sum8(eb_ref[...])

    @pl.when(i == GRID - 1)
    def _finish():
        sn = jnp.sum(acc_ref[0:8, :], axis=0, keepdims=True)
        se = jnp.sum(acc_ref[8:16, :] + acc_ref[16:24, :], axis=0, keepdims=True)
        g = g_ref[...]
        h = (jnp.dot(sn, w1_ref[0:HIDDEN, :], preferred_element_type=jnp.float32)
             + jnp.dot(se, w1_ref[HIDDEN:2 * HIDDEN, :], preferred_element_type=jnp.float32)
             + jnp.dot(g, w1_ref[2 * HIDDEN:3 * HIDDEN, :], preferred_element_type=jnp.float32)
             + b1_ref[...])
        h = jnp.maximum(h, 0.0)
        out = jnp.dot(h, w2_ref[...], preferred_element_type=jnp.float32) + b2_ref[...]
        mean = jnp.mean(out, axis=-1, keepdims=True)
        var = jnp.mean((out - mean) ** 2, axis=-1, keepdims=True)
        out_ref[...] = ((out - mean) * jax.lax.rsqrt(var + 1e-5)
                        * gamma_ref[...] + beta_ref[...])


def kernel(x, edge_attr_updated, global_attr, W1, b1, W2, b2, gamma, beta):
    b1r = b1.reshape(1, HIDDEN)
    b2r = b2.reshape(1, HIDDEN)
    gammar = gamma.reshape(1, HIDDEN)
    betar = beta.reshape(1, HIDDEN)

    const = lambda i: (0, 0)
    return pl.pallas_call(
        _gb_kernel,
        grid=(GRID,),
        in_specs=[
            pl.BlockSpec((BX, HIDDEN), lambda i: (i, 0)),
            pl.BlockSpec((BE, HIDDEN), lambda i: (2 * i, 0)),
            pl.BlockSpec((BE, HIDDEN), lambda i: (2 * i + 1, 0)),
            pl.BlockSpec((1, HIDDEN), const),
            pl.BlockSpec((3 * HIDDEN, HIDDEN), const),
            pl.BlockSpec((1, HIDDEN), const),
            pl.BlockSpec((HIDDEN, HIDDEN), const),
            pl.BlockSpec((1, HIDDEN), const),
            pl.BlockSpec((1, HIDDEN), const),
            pl.BlockSpec((1, HIDDEN), const),
        ],
        out_specs=pl.BlockSpec((1, HIDDEN), const),
        out_shape=jax.ShapeDtypeStruct((1, HIDDEN), jnp.float32),
        scratch_shapes=[pltpu.VMEM((24, HIDDEN), jnp.float32)],
        compiler_params=pltpu.CompilerParams(
            dimension_semantics=("arbitrary",),
        ),
    )(x, edge_attr_updated, edge_attr_updated, global_attr, W1, b1r, W2,
      b2r, gammar, betar)


# speedup vs baseline: 1.0068x; 1.0004x over previous
import jax
import jax.numpy as jnp
from jax.experimental import pallas as pl
from jax.experimental.pallas import tpu as pltpu

HIDDEN = 128
N_EDGES = 320000
N_X = 10000
CH = 6400
NSTEP = N_EDGES // CH
NBUF = 4
XCH = 5000
XBUF = 2


def _tree_sum8(a):
    rows = a.shape[0]
    if rows > 128 and rows % 128 == 0:
        a = a.reshape(rows // 128, 128, HIDDEN).sum(axis=0)
        rows = 128
    while rows > 8 and rows % 16 == 0:
        rows //= 2
        a = a[:rows] + a[rows:]
    if rows > 8:
        a = a.reshape(rows // 8, 8, HIDDEN).sum(axis=0)
    return a


def _kern(x_hbm, e_hbm, g_ref, w1_ref, b1_ref, w2_ref, b2_ref,
          gamma_ref, beta_ref, out_ref,
          b0, b1s, b2s, b3, x0, x1, se0, se1, se2, se3, sx0, sx1):
    ebufs = (b0, b1s, b2s, b3)
    esems = (se0, se1, se2, se3)
    xbufs = (x0, x1)
    xsems = (sx0, sx1)

    def ecopy(t):
        return pltpu.make_async_copy(
            e_hbm.at[pl.ds(t * CH, CH)], ebufs[t % NBUF], esems[t % NBUF])

    def xcopy(t):
        return pltpu.make_async_copy(
            x_hbm.at[pl.ds(t * XCH, XCH)], xbufs[t], xsems[t])

    for t in range(NBUF):
        ecopy(t).start()
    xcopy(0).start()
    xcopy(1).start()

    acc_e = jnp.zeros((8, HIDDEN), jnp.float32)
    for t in range(NSTEP):
        ecopy(t).wait()
        acc_e = acc_e + _tree_sum8(ebufs[t % NBUF][...])
        if t + NBUF < NSTEP:
            ecopy(t + NBUF).start()

    acc_x = jnp.zeros((8, HIDDEN), jnp.float32)
    for t in range(XBUF):
        xcopy(t).wait()
        acc_x = acc_x + _tree_sum8(
            xbufs[t][...].reshape(XCH // 200, 200, HIDDEN).sum(axis=0))

    sn = jnp.sum(acc_x, axis=0, keepdims=True)
    se = jnp.sum(acc_e, axis=0, keepdims=True)
    g = g_ref[...]
    h = (jnp.dot(sn, w1_ref[0:HIDDEN, :], preferred_element_type=jnp.float32)
         + jnp.dot(se, w1_ref[HIDDEN:2 * HIDDEN, :], preferred_element_type=jnp.float32)
         + jnp.dot(g, w1_ref[2 * HIDDEN:3 * HIDDEN, :], preferred_element_type=jnp.float32)
         + b1_ref[...])
    h = jnp.maximum(h, 0.0)
    out = jnp.dot(h, w2_ref[...], preferred_element_type=jnp.float32) + b2_ref[...]
    mean = jnp.mean(out, axis=-1, keepdims=True)
    var = jnp.mean((out - mean) ** 2, axis=-1, keepdims=True)
    out_ref[...] = ((out - mean) * jax.lax.rsqrt(var + 1e-5)
                    * gamma_ref[...] + beta_ref[...])


def kernel(x, edge_attr_updated, global_attr, W1, b1, W2, b2, gamma, beta):
    vm = lambda: pl.BlockSpec(memory_space=pl.ANY)
    return pl.pallas_call(
        _kern,
        in_specs=[
            vm(), vm(),
            pl.BlockSpec((1, HIDDEN), lambda: (0, 0)),
            pl.BlockSpec((3 * HIDDEN, HIDDEN), lambda: (0, 0)),
            pl.BlockSpec((1, HIDDEN), lambda: (0, 0)),
            pl.BlockSpec((HIDDEN, HIDDEN), lambda: (0, 0)),
            pl.BlockSpec((1, HIDDEN), lambda: (0, 0)),
            pl.BlockSpec((1, HIDDEN), lambda: (0, 0)),
            pl.BlockSpec((1, HIDDEN), lambda: (0, 0)),
        ],
        out_specs=pl.BlockSpec((1, HIDDEN), lambda: (0, 0)),
        out_shape=jax.ShapeDtypeStruct((1, HIDDEN), jnp.float32),
        scratch_shapes=[
            pltpu.VMEM((CH, HIDDEN), jnp.float32),
            pltpu.VMEM((CH, HIDDEN), jnp.float32),
            pltpu.VMEM((CH, HIDDEN), jnp.float32),
            pltpu.VMEM((CH, HIDDEN), jnp.float32),
            pltpu.VMEM((XCH, HIDDEN), jnp.float32),
            pltpu.VMEM((XCH, HIDDEN), jnp.float32),
            pltpu.SemaphoreType.DMA,
            pltpu.SemaphoreType.DMA,
            pltpu.SemaphoreType.DMA,
            pltpu.SemaphoreType.DMA,
            pltpu.SemaphoreType.DMA,
            pltpu.SemaphoreType.DMA,
        ],
    )(x, edge_attr_updated, global_attr, W1, b1.reshape(1, HIDDEN), W2,
      b2.reshape(1, HIDDEN), gamma.reshape(1, HIDDEN), beta.reshape(1, HIDDEN))


# speedup vs baseline: 1.0081x; 1.0013x over previous
import jax
import jax.numpy as jnp
from jax.experimental import pallas as pl
from jax.experimental.pallas import tpu as pltpu

HIDDEN = 128
N_EDGES = 320000
N_X = 10000
GRID = 25
HALF_BLOCKS = GRID
BE = N_EDGES // (2 * GRID)
BX = N_X // GRID


def _tree_sum8(a):
    rows = a.shape[0]
    if rows > 128 and rows % 128 == 0:
        a = a.reshape(rows // 128, 128, HIDDEN).sum(axis=0)
        rows = 128
    while rows > 8 and rows % 16 == 0:
        rows //= 2
        a = a[:rows] + a[rows:]
    if rows > 8:
        a = a.reshape(rows // 8, 8, HIDDEN).sum(axis=0)
    return a


def _gb_kernel(x_ref, ea_ref, eb_ref, g_ref, w1_ref, b1_ref, w2_ref, b2_ref,
               gamma_ref, beta_ref, out_ref, acc_ref):
    i = pl.program_id(0)

    @pl.when(i == 0)
    def _init():
        acc_ref[...] = jnp.zeros_like(acc_ref)

    acc_ref[0:8, :] += _tree_sum8(x_ref[...])
    acc_ref[8:16, :] += _tree_sum8(ea_ref[...])
    acc_ref[16:24, :] += _tree_sum8(eb_ref[...])

    @pl.when(i == GRID - 1)
    def _finish():
        sn = jnp.sum(acc_ref[0:8, :], axis=0, keepdims=True)
        se = jnp.sum(acc_ref[8:16, :] + acc_ref[16:24, :], axis=0, keepdims=True)
        g = g_ref[...]
        h = (jnp.dot(sn, w1_ref[0:HIDDEN, :], preferred_element_type=jnp.float32)
             + jnp.dot(se, w1_ref[HIDDEN:2 * HIDDEN, :], preferred_element_type=jnp.float32)
             + jnp.dot(g, w1_ref[2 * HIDDEN:3 * HIDDEN, :], preferred_element_type=jnp.float32)
             + b1_ref[...])
        h = jnp.maximum(h, 0.0)
        out = jnp.dot(h, w2_ref[...], preferred_element_type=jnp.float32) + b2_ref[...]
        mean = jnp.mean(out, axis=-1, keepdims=True)
        var = jnp.mean((out - mean) ** 2, axis=-1, keepdims=True)
        out_ref[...] = ((out - mean) * jax.lax.rsqrt(var + 1e-5)
                        * gamma_ref[...] + beta_ref[...])


def kernel(x, edge_attr_updated, global_attr, W1, b1, W2, b2, gamma, beta):
    b1r = b1.reshape(1, HIDDEN)
    b2r = b2.reshape(1, HIDDEN)
    gammar = gamma.reshape(1, HIDDEN)
    betar = beta.reshape(1, HIDDEN)

    const = lambda i: (0, 0)
    return pl.pallas_call(
        _gb_kernel,
        grid=(GRID,),
        in_specs=[
            pl.BlockSpec((BX, HIDDEN), lambda i: (i, 0)),
            pl.BlockSpec((BE, HIDDEN), lambda i: (2 * i, 0)),
            pl.BlockSpec((BE, HIDDEN), lambda i: (2 * i + 1, 0)),
            pl.BlockSpec((1, HIDDEN), const),
            pl.BlockSpec((3 * HIDDEN, HIDDEN), const),
            pl.BlockSpec((1, HIDDEN), const),
            pl.BlockSpec((HIDDEN, HIDDEN), const),
            pl.BlockSpec((1, HIDDEN), const),
            pl.BlockSpec((1, HIDDEN), const),
            pl.BlockSpec((1, HIDDEN), const),
        ],
        out_specs=pl.BlockSpec((1, HIDDEN), const),
        out_shape=jax.ShapeDtypeStruct((1, HIDDEN), jnp.float32),
        scratch_shapes=[pltpu.VMEM((24, HIDDEN), jnp.float32)],
        compiler_params=pltpu.CompilerParams(
            dimension_semantics=("arbitrary",),
        ),
    )(x, edge_attr_updated, edge_attr_updated, global_attr, W1, b1r, W2,
      b2r, gammar, betar)


# speedup vs baseline: 1.0091x; 1.0009x over previous
import jax
import jax.numpy as jnp
from jax.experimental import pallas as pl
from jax.experimental.pallas import tpu as pltpu

HIDDEN = 128
N_EDGES = 320000
N_X = 10000
GRID = 25
BE = N_EDGES // (2 * GRID)
BX = N_X // GRID


def _tree_sum8(a):
    rows = a.shape[0]
    if rows > 128 and rows % 128 == 0:
        a = a.reshape(rows // 128, 128, HIDDEN).sum(axis=0)
        rows = 128
    while rows > 8 and rows % 16 == 0:
        rows //= 2
        a = a[:rows] + a[rows:]
    if rows > 8:
        a = a.reshape(rows // 8, 8, HIDDEN).sum(axis=0)
    return a


def _gb_kernel(x_ref, ea_ref, eb_ref, g_ref, w1_ref, b1_ref, w2_ref, b2_ref,
               gamma_ref, beta_ref, out_ref, acc_ref):
    i = pl.program_id(0)

    @pl.when(i == 0)
    def _init():
        acc_ref[...] = jnp.zeros_like(acc_ref)

    acc_ref[0:8, :] += _tree_sum8(x_ref[...])
    acc_ref[8:16, :] += _tree_sum8(ea_ref[...])
    acc_ref[16:24, :] += _tree_sum8(eb_ref[...])

    @pl.when(i == GRID - 1)
    def _finish():
        sn = jnp.sum(acc_ref[0:8, :], axis=0, keepdims=True)
        se = jnp.sum(acc_ref[8:16, :] + acc_ref[16:24, :], axis=0, keepdims=True)
        g = g_ref[...]
        h = (jnp.dot(sn, w1_ref[0:HIDDEN, :], preferred_element_type=jnp.float32)
             + jnp.dot(se, w1_ref[HIDDEN:2 * HIDDEN, :], preferred_element_type=jnp.float32)
             + jnp.dot(g, w1_ref[2 * HIDDEN:3 * HIDDEN, :], preferred_element_type=jnp.float32)
             + b1_ref[...])
        h = jnp.maximum(h, 0.0)
        out = jnp.dot(h, w2_ref[...], preferred_element_type=jnp.float32) + b2_ref[...]
        mean = jnp.mean(out, axis=-1, keepdims=True)
        var = jnp.mean((out - mean) ** 2, axis=-1, keepdims=True)
        out_ref[...] = ((out - mean) * jax.lax.rsqrt(var + 1e-5)
                        * gamma_ref[...] + beta_ref[...])


def kernel(x, edge_attr_updated, global_attr, W1, b1, W2, b2, gamma, beta):
    b1r = b1.reshape(1, HIDDEN)
    b2r = b2.reshape(1, HIDDEN)
    gammar = gamma.reshape(1, HIDDEN)
    betar = beta.reshape(1, HIDDEN)

    const = lambda i: (0, 0)
    return pl.pallas_call(
        _gb_kernel,
        grid=(GRID,),
        in_specs=[
            pl.BlockSpec((BX, HIDDEN), lambda i: (i, 0)),
            pl.BlockSpec((BE, HIDDEN), lambda i: (2 * i, 0)),
            pl.BlockSpec((BE, HIDDEN), lambda i: (2 * i + 1, 0)),
            pl.BlockSpec((1, HIDDEN), const),
            pl.BlockSpec((3 * HIDDEN, HIDDEN), const),
            pl.BlockSpec((1, HIDDEN), const),
            pl.BlockSpec((HIDDEN, HIDDEN), const),
            pl.BlockSpec((1, HIDDEN), const),
            pl.BlockSpec((1, HIDDEN), const),
            pl.BlockSpec((1, HIDDEN), const),
        ],
        out_specs=pl.BlockSpec((1, HIDDEN), const),
        out_shape=jax.ShapeDtypeStruct((1, HIDDEN), jnp.float32),
        scratch_shapes=[pltpu.VMEM((24, HIDDEN), jnp.float32)],
        compiler_params=pltpu.CompilerParams(
            dimension_semantics=("arbitrary",),
        ),
    )(x, edge_attr_updated, edge_attr_updated, global_attr, W1, b1r, W2,
      b2r, gammar, betar)
